# baseline (speedup 1.0000x reference)
import functools

import jax
import jax.numpy as jnp
from jax import lax
from jax.experimental import pallas as pl
from jax.experimental.pallas import tpu as pltpu
from jax.experimental.pallas import tpu_sc as plsc

N = 10000
E = 320000
D = 128
LANES = 16
CHUNK = 128
NC = 2
NS = 16
NW = NC * NS
ITERS = 80
PADCHUNKS = ITERS * NW
E_PAD = PADCHUNKS * CHUNK
RBLK = 80
NRBLK = N // RBLK
RITERS = -(-NRBLK // NS)


def _mm_body(x_ref, w_ref, b_ref, o_ref):
    o_ref[...] = (
        jnp.dot(x_ref[...], w_ref[...], preferred_element_type=jnp.float32)
        + b_ref[...]
    )


def _matmul_bias(x, W, b):
    M = x.shape[0]
    BM = 1000
    return pl.pallas_call(
        _mm_body,
        grid=(M // BM,),
        in_specs=[
            pl.BlockSpec((BM, D), lambda i: (i, 0)),
            pl.BlockSpec((D, D), lambda i: (0, 0)),
            pl.BlockSpec((1, D), lambda i: (0, 0)),
        ],
        out_specs=pl.BlockSpec((BM, D), lambda i: (i, 0)),
        out_shape=jax.ShapeDtypeStruct((M, D), jnp.float32),
    )(x, W, b.reshape(1, D))


def _add_body(a_ref, b_ref, o_ref):
    o_ref[...] = a_ref[...] + b_ref[...]


def _add2(a, b):
    BM = 1000
    return pl.pallas_call(
        _add_body,
        grid=(N // BM,),
        in_specs=[pl.BlockSpec((BM, D), lambda i: (i, 0))] * 2,
        out_specs=pl.BlockSpec((BM, D), lambda i: (i, 0)),
        out_shape=jax.ShapeDtypeStruct((N, D), jnp.float32),
    )(a, b)


def _sc_scatter(h, edata):
    mesh = plsc.VectorSubcoreMesh(core_axis_name="c", subcore_axis_name="s")

    @functools.partial(
        pl.kernel,
        out_type=jax.ShapeDtypeStruct((NC, N, D), jnp.float32),
        mesh=mesh,
        compiler_params=pltpu.CompilerParams(needs_layout_passes=False),
        scratch_types=(
            pltpu.VMEM((3, CHUNK), jnp.int32),
            pltpu.VMEM((CHUNK,), jnp.int32),
            pltpu.VMEM((CHUNK, D), jnp.float32),
            pltpu.VMEM_SHARED((N, D), jnp.float32),
        ),
    )
    def k(h_hbm, edata_hbm, out_hbm, ed, vv_v, msgs, acc):
        cid = lax.axis_index("c")
        sid = lax.axis_index("s")
        w = sid * NC + cid

        def zero_body(r, carry):
            for j in range(D // LANES):
                msgs[r, pl.ds(j * LANES, LANES)] = jnp.zeros(
                    (LANES,), jnp.float32)
            return carry

        lax.fori_loop(0, RBLK, zero_body, 0)
        for t in range(RITERS):
            rb = sid + t * NS

            @pl.when(rb < NRBLK)
            def _():
                r0 = pl.multiple_of(rb * RBLK, 8)
                pltpu.sync_copy(
                    msgs.at[pl.ds(0, RBLK)], acc.at[pl.ds(r0, RBLK)])

        plsc.subcore_barrier()

        def chunk_body(i, carry):
            pltpu.sync_copy(edata_hbm.at[w + i * NW], ed)
            pltpu.sync_copy(h_hbm.at[ed.at[1]], msgs)
            for j in range(CHUNK // LANES):
                sl = pl.ds(j * LANES, LANES)
                vv_v[sl] = ed[2, sl]

            def scale_body(g, carry2):
                for u in range(4):
                    e = g * 4 + u
                    v = plsc.bitcast(
                        plsc.load_gather(
                            vv_v, [jnp.full((LANES,), e, jnp.int32)]),
                        jnp.float32)
                    for j in range(D // LANES):
                        fsl = pl.ds(j * LANES, LANES)
                        msgs[e, fsl] = msgs[e, fsl] * v
                return carry2

            lax.fori_loop(0, CHUNK // 4, scale_body, 0)
            pltpu.sync_copy(msgs, acc.at[ed.at[0]], add=True)
            return carry

        lax.fori_loop(0, ITERS, chunk_body, 0)

        plsc.subcore_barrier()

        for t in range(RITERS):
            rb = sid + t * NS

            @pl.when(rb < NRBLK)
            def _():
                r0 = pl.multiple_of(rb * RBLK, 8)
                pltpu.sync_copy(
                    acc.at[pl.ds(r0, RBLK)],
                    out_hbm.at[cid, pl.ds(r0, RBLK)],
                )

    return k(h, edata)


def kernel(x, adj_indices, adj_values, W, b):
    h = _matmul_bias(x, W, b)
    pad = E_PAD - E
    rows = jnp.pad(adj_indices[0], (0, pad))
    cols = jnp.pad(adj_indices[1], (0, pad))
    vals = jnp.pad(adj_values, (0, pad))
    edata = jnp.stack(
        [rows.reshape(PADCHUNKS, CHUNK),
         cols.reshape(PADCHUNKS, CHUNK),
         lax.bitcast_convert_type(vals, jnp.int32).reshape(PADCHUNKS, CHUNK)],
        axis=1)
    parts = _sc_scatter(h, edata)
    return _add2(parts[0], parts[1])


# speedup vs baseline: 1.0668x; 1.0668x over previous
import functools

import jax
import jax.numpy as jnp
from jax import lax
from jax.experimental import pallas as pl
from jax.experimental.pallas import tpu as pltpu
from jax.experimental.pallas import tpu_sc as plsc

N = 10000
E = 320000
D = 128
LANES = 16
CHUNK = 128
NC = 2
NS = 16
NW = NC * NS
ITERS = 80
PADCHUNKS = ITERS * NW
E_PAD = PADCHUNKS * CHUNK
RBLK = 80
NRBLK = N // RBLK
RITERS = -(-NRBLK // NS)


def _mm_body(x_ref, w_ref, b_ref, o_ref):
    o_ref[...] = (
        jnp.dot(x_ref[...], w_ref[...], preferred_element_type=jnp.float32)
        + b_ref[...]
    )


def _matmul_bias(x, W, b):
    M = x.shape[0]
    BM = 1000
    return pl.pallas_call(
        _mm_body,
        grid=(M // BM,),
        in_specs=[
            pl.BlockSpec((BM, D), lambda i: (i, 0)),
            pl.BlockSpec((D, D), lambda i: (0, 0)),
            pl.BlockSpec((1, D), lambda i: (0, 0)),
        ],
        out_specs=pl.BlockSpec((BM, D), lambda i: (i, 0)),
        out_shape=jax.ShapeDtypeStruct((M, D), jnp.float32),
    )(x, W, b.reshape(1, D))


def _add_body(a_ref, b_ref, o_ref):
    o_ref[...] = a_ref[...] + b_ref[...]


def _add2(a, b):
    BM = 1000
    return pl.pallas_call(
        _add_body,
        grid=(N // BM,),
        in_specs=[pl.BlockSpec((BM, D), lambda i: (i, 0))] * 2,
        out_specs=pl.BlockSpec((BM, D), lambda i: (i, 0)),
        out_shape=jax.ShapeDtypeStruct((N, D), jnp.float32),
    )(a, b)


def _sc_scatter(h, edata):
    mesh = plsc.VectorSubcoreMesh(core_axis_name="c", subcore_axis_name="s")

    @functools.partial(
        pl.kernel,
        out_type=jax.ShapeDtypeStruct((NC, N, D), jnp.float32),
        mesh=mesh,
        compiler_params=pltpu.CompilerParams(needs_layout_passes=False),
        scratch_types=(
            pltpu.VMEM((3, CHUNK), jnp.int32),
            pltpu.VMEM((CHUNK,), jnp.int32),
            pltpu.VMEM((CHUNK, D), jnp.float32),
            pltpu.VMEM_SHARED((N, D), jnp.float32),
        ),
    )
    def k(h_hbm, edata_hbm, out_hbm, ed, vv_v, msgs, acc):
        cid = lax.axis_index("c")
        sid = lax.axis_index("s")
        w = sid * NC + cid

        @plsc.parallel_loop(0, RBLK)
        def _zero(r):
            for j in range(D // LANES):
                msgs[r, pl.ds(j * LANES, LANES)] = jnp.zeros(
                    (LANES,), jnp.float32)
        for t in range(RITERS):
            rb = sid + t * NS

            @pl.when(rb < NRBLK)
            def _():
                r0 = pl.multiple_of(rb * RBLK, 8)
                pltpu.sync_copy(
                    msgs.at[pl.ds(0, RBLK)], acc.at[pl.ds(r0, RBLK)])

        plsc.subcore_barrier()

        def chunk_body(i, carry):
            pltpu.sync_copy(edata_hbm.at[w + i * NW], ed)
            pltpu.sync_copy(h_hbm.at[ed.at[1]], msgs)
            for j in range(CHUNK // LANES):
                sl = pl.ds(j * LANES, LANES)
                vv_v[sl] = ed[2, sl]

            @plsc.parallel_loop(0, CHUNK, unroll=4)
            def _scale(e):
                v = plsc.bitcast(
                    plsc.load_gather(
                        vv_v, [jnp.full((LANES,), e, jnp.int32)]),
                    jnp.float32)
                for j in range(D // LANES):
                    fsl = pl.ds(j * LANES, LANES)
                    msgs[e, fsl] = msgs[e, fsl] * v
            pltpu.sync_copy(msgs, acc.at[ed.at[0]], add=True)
            return carry

        lax.fori_loop(0, ITERS, chunk_body, 0)

        plsc.subcore_barrier()

        for t in range(RITERS):
            rb = sid + t * NS

            @pl.when(rb < NRBLK)
            def _():
                r0 = pl.multiple_of(rb * RBLK, 8)
                pltpu.sync_copy(
                    acc.at[pl.ds(r0, RBLK)],
                    out_hbm.at[cid, pl.ds(r0, RBLK)],
                )

    return k(h, edata)


def kernel(x, adj_indices, adj_values, W, b):
    h = _matmul_bias(x, W, b)
    pad = E_PAD - E
    rows = jnp.pad(adj_indices[0], (0, pad))
    cols = jnp.pad(adj_indices[1], (0, pad))
    vals = jnp.pad(adj_values, (0, pad))
    edata = jnp.stack(
        [rows.reshape(PADCHUNKS, CHUNK),
         cols.reshape(PADCHUNKS, CHUNK),
         lax.bitcast_convert_type(vals, jnp.int32).reshape(PADCHUNKS, CHUNK)],
        axis=1)
    parts = _sc_scatter(h, edata)
    return _add2(parts[0], parts[1])
